# Initial kernel scaffold; baseline (speedup 1.0000x reference)
import jax
import jax.numpy as jnp
from jax.experimental import pallas as pl


def kernel(x, edge_index, W1, b1, W2, b2):
    raise NotImplementedError("write your pallas kernel here")


# speedup vs baseline: 5.7609x; 5.7609x over previous
import functools

import jax
import jax.numpy as jnp
from jax import lax
from jax.experimental import pallas as pl
from jax.experimental.pallas import tpu as pltpu
from jax.experimental.pallas import tpu_sc as plsc

NSUB = 16
NCORE = 2
EBLK = 128
F2 = 64
ROWT = 256

_HI = jax.lax.Precision.HIGHEST


def _make_aprop(nblk: int, npad: int):
    slab = npad + 16
    rows_per_sub = npad // NSUB
    nchunk = rows_per_sub // 128
    mesh = plsc.VectorSubcoreMesh(core_axis_name="c", subcore_axis_name="s")

    @functools.partial(
        pl.kernel,
        out_type=jax.ShapeDtypeStruct((NCORE, npad, F2), jnp.float32),
        mesh=mesh,
        scratch_types=[
            pltpu.VMEM((nblk, EBLK), jnp.int32),
            pltpu.VMEM((nblk, EBLK), jnp.int32),
            pltpu.VMEM((EBLK, F2), jnp.float32),
            pltpu.VMEM_SHARED((slab, F2), jnp.float32),
            pltpu.SemaphoreType.DMA,
        ],
        compiler_params=pltpu.CompilerParams(use_tc_tiling_on_sc=False),
    )
    def aprop(in_hbm, src_hbm, dst_hbm, zero_hbm, out_hbm,
              src_v, dst_v, gbuf, acc, sem):
        c = lax.axis_index("c")
        s = lax.axis_index("s")
        pltpu.sync_copy(src_hbm.at[s], src_v)
        pltpu.sync_copy(dst_hbm.at[s], dst_v)
        pltpu.sync_copy(zero_hbm, gbuf)
        base = s * rows_per_sub
        for k in range(nchunk):
            pltpu.sync_copy(gbuf, acc.at[pl.ds(base + k * 128, 128)])

        @pl.when(s == NSUB - 1)
        def _():
            pltpu.sync_copy(gbuf.at[pl.ds(0, 16)], acc.at[pl.ds(npad, 16)])

        plsc.subcore_barrier()

        def run(in_h, out_h):
            def body(j, carry):
                pltpu.async_copy(in_h.at[src_v.at[j]], gbuf, sem).wait()
                pltpu.sync_copy(gbuf, acc.at[dst_v.at[j]], add=True)
                return carry

            lax.fori_loop(0, nblk, body, 0)
            plsc.subcore_barrier()
            for k in range(nchunk):
                r = base + k * 128
                pltpu.sync_copy(acc.at[pl.ds(r, 128)], out_h.at[pl.ds(r, 128)])

        @pl.when(c == 0)
        def _():
            run(in_hbm.at[0], out_hbm.at[0])

        @pl.when(c == 1)
        def _():
            run(in_hbm.at[1], out_hbm.at[1])

    return aprop


def _rowscale_split(a, svec, npad):
    F = a.shape[1]

    def body(a_ref, s_ref, o_ref):
        av = a_ref[...] * s_ref[...]
        o_ref[0] = av[:, :F2]
        o_ref[1] = av[:, F2:]

    return pl.pallas_call(
        body,
        grid=(npad // ROWT,),
        in_specs=[
            pl.BlockSpec((ROWT, F), lambda i: (i, 0)),
            pl.BlockSpec((ROWT, 1), lambda i: (i, 0)),
        ],
        out_specs=pl.BlockSpec((2, ROWT, F2), lambda i: (0, i, 0)),
        out_shape=jax.ShapeDtypeStruct((2, npad, F2), jnp.float32),
    )(a, svec)


def _rowscale_stacked(v, svec, npad):
    def body(v_ref, s_ref, o_ref):
        o_ref[...] = v_ref[...] * s_ref[...][None]

    return pl.pallas_call(
        body,
        grid=(npad // ROWT,),
        in_specs=[
            pl.BlockSpec((2, ROWT, F2), lambda i: (0, i, 0)),
            pl.BlockSpec((ROWT, 1), lambda i: (i, 0)),
        ],
        out_specs=pl.BlockSpec((2, ROWT, F2), lambda i: (0, i, 0)),
        out_shape=jax.ShapeDtypeStruct((2, npad, F2), jnp.float32),
    )(v, svec)


def _cheb_mix(t, v1_parts, v2_parts, svec, W, b, npad, emit_next):
    Fin = t.shape[1]
    H = W.shape[2]
    nparts = len(v1_parts)
    ngroups = H // (2 * F2)
    b2d = b.reshape(1, H)

    def body(*refs):
        t_ref = refs[0]
        v1_refs = refs[1:1 + nparts]
        v2_refs = refs[1 + nparts:1 + 2 * nparts]
        s_ref, w_ref, b_ref = refs[1 + 2 * nparts:4 + 2 * nparts]
        out_refs = refs[4 + 2 * nparts:]
        sv = s_ref[...]
        tt = t_ref[...]
        v1c = jnp.concatenate(
            [r[k] for r in v1_refs for k in range(2)], axis=1)
        v2c = jnp.concatenate(
            [r[k] for r in v2_refs for k in range(2)], axis=1)
        w = w_ref[...]
        acc = jnp.dot(tt, w[0], precision=_HI, preferred_element_type=jnp.float32)
        acc = acc - jnp.dot(sv * v1c, w[1], precision=_HI,
                            preferred_element_type=jnp.float32)
        acc = acc + jnp.dot(2.0 * (sv * v2c) - tt, w[2], precision=_HI,
                            preferred_element_type=jnp.float32)
        h = jnp.maximum(acc + b_ref[...], 0.0)
        out_refs[0][...] = h
        if emit_next:
            u = sv * h
            for g in range(ngroups):
                for k in range(2):
                    lo = (2 * g + k) * F2
                    out_refs[1 + g][k] = u[:, lo:lo + F2]

    part_spec = pl.BlockSpec((2, ROWT, F2), lambda i: (0, i, 0))
    in_specs = [pl.BlockSpec((ROWT, Fin), lambda i: (i, 0))]
    in_specs += [part_spec] * (2 * nparts)
    in_specs += [
        pl.BlockSpec((ROWT, 1), lambda i: (i, 0)),
        pl.BlockSpec(W.shape, lambda i: (0, 0, 0)),
        pl.BlockSpec((1, H), lambda i: (0, 0)),
    ]
    out_shape = [jax.ShapeDtypeStruct((npad, H), jnp.float32)]
    out_specs = [pl.BlockSpec((ROWT, H), lambda i: (i, 0))]
    if emit_next:
        for _ in range(ngroups):
            out_shape.append(
                jax.ShapeDtypeStruct((2, npad, F2), jnp.float32))
            out_specs.append(part_spec)

    res = pl.pallas_call(
        body,
        grid=(npad // ROWT,),
        in_specs=in_specs,
        out_specs=out_specs,
        out_shape=out_shape,
    )(t, *v1_parts, *v2_parts, svec, W, b2d)
    return res if emit_next else res[0]


def kernel(x, edge_index, W1, b1, W2, b2):
    N, IN = x.shape
    H = W1.shape[2]
    E = edge_index.shape[1]

    npad = ((N + 2047) // 2048) * 2048
    nblk = -(-E // (NSUB * EBLK))
    ep = NSUB * nblk * EBLK

    row = edge_index[0]
    col = edge_index[1]
    pad = ep - E
    zi = jnp.zeros((pad,), jnp.int32)
    ti = jnp.full((pad,), npad, jnp.int32)
    src_p = jnp.concatenate([row, zi]).reshape(NSUB, nblk, EBLK)
    dst_p = jnp.concatenate([col, ti]).reshape(NSUB, nblk, EBLK)
    src_d = jnp.concatenate([col, zi]).reshape(NSUB, nblk, EBLK)
    dst_d = jnp.concatenate([row, ti]).reshape(NSUB, nblk, EBLK)

    xp = jnp.zeros((npad, IN), jnp.float32).at[:N].set(x)

    zbuf = jnp.zeros((EBLK, F2), jnp.float32)
    aprop = _make_aprop(nblk, npad)

    ones_in = jnp.ones((NCORE, npad, F2), jnp.float32)
    degout = aprop(ones_in, src_d, dst_d, zbuf)
    deg = degout[0, :, 0]

    s = jnp.where(deg > 0, jax.lax.rsqrt(jnp.where(deg > 0, deg, 1.0)), 0.0)
    sc = s.reshape(npad, 1)
    s2c = (s * s).reshape(npad, 1)

    u0 = _rowscale_split(xp, sc, npad)
    v1 = aprop(u0, src_p, dst_p, zbuf)
    u1 = _rowscale_stacked(v1, s2c, npad)
    v2 = aprop(u1, src_p, dst_p, zbuf)
    h, uA, uB = _cheb_mix(xp, [v1], [v2], sc, W1, b1, npad, True)

    vA1 = aprop(uA, src_p, dst_p, zbuf)
    vB1 = aprop(uB, src_p, dst_p, zbuf)
    uA1 = _rowscale_stacked(vA1, s2c, npad)
    uB1 = _rowscale_stacked(vB1, s2c, npad)
    vA2 = aprop(uA1, src_p, dst_p, zbuf)
    vB2 = aprop(uB1, src_p, dst_p, zbuf)
    out = _cheb_mix(h, [vA1, vB1], [vA2, vB2], sc, W2, b2, npad, False)

    return out[:N]
